# baseline (speedup 1.0000x reference)
import functools

import jax
import jax.numpy as jnp
from jax import lax
from jax.experimental import pallas as pl
from jax.experimental.pallas import tpu as pltpu
from jax.experimental.pallas import tpu_sc as plsc

N = 50000
E = 800000
DM = 64
L = 6

NS = 16
CHUNK = 128

SIB = 28
NSIB = 14
CPT = SIB * NSIB
EPT = CPT * CHUNK
EPAD = NS * EPT

T = 25024
AROWS = 25088
AZ = AROWS // NS
CRPT = T // NS
NOUT = 2 * T

IB = 24
CAPC = 416
FB = 1024
DROW = T

ROWBLK = 2000
GRID = N // ROWBLK

_f32 = jnp.float32
_i32 = jnp.int32

_SC_PARAMS = pltpu.CompilerParams(use_tc_tiling_on_sc=False,
                                  needs_layout_passes=False)


def _part_body(srcp, dstp, psrc, pdst, cnts,
               src_st, dst_st, sb_src, sb_dst, d_src, d_dst, cnt_v):
    cid = lax.axis_index("c")
    sid = lax.axis_index("s")
    base_lo = cid * T

    for t in range(CHUNK // 16):
        d_src[pl.ds(t * 16, 16)] = jnp.zeros((16,), _i32)
        d_dst[pl.ds(t * 16, 16)] = jnp.full((16,), DROW, _i32)

    def scan_stage(s, carry):
        w, nch = carry
        pltpu.sync_copy(srcp.at[sid, pl.ds(s * SIB, SIB)], src_st)
        pltpu.sync_copy(dstp.at[sid, pl.ds(s * SIB, SIB)], dst_st)

        def vreg(i, carry):
            w, nch = carry
            j = i // 8
            o = (i % 8) * 16
            s16 = src_st[j, pl.ds(o, 16)]
            d16 = dst_st[j, pl.ds(o, 16)] - base_lo
            mask = (d16 >= 0) & (d16 < T)
            c = jnp.max(plsc.all_reduce_population_count(mask))
            plsc.store_compressed(sb_src.at[pl.ds(w, 16)], s16, mask=mask)
            plsc.store_compressed(sb_dst.at[pl.ds(w, 16)], d16, mask=mask)
            w = w + c
            do_flush = w >= FB

            @pl.when(do_flush)
            def _():
                def one(f, carry):
                    pltpu.sync_copy(sb_src.at[pl.ds(f * CHUNK, CHUNK)],
                                    psrc.at[cid, sid, nch + f])
                    pltpu.sync_copy(sb_dst.at[pl.ds(f * CHUNK, CHUNK)],
                                    pdst.at[cid, sid, nch + f])
                    return carry
                lax.fori_loop(0, FB // CHUNK, one, 0)
                sb_src[pl.ds(0, 16)] = sb_src[pl.ds(FB, 16)]
                sb_dst[pl.ds(0, 16)] = sb_dst[pl.ds(FB, 16)]

            fl = do_flush.astype(_i32)
            return w - FB * fl, nch + (FB // CHUNK) * fl
        return lax.fori_loop(0, SIB * 8, vreg, (w, nch))

    w, nch = lax.fori_loop(0, NSIB, scan_stage,
                           (jnp.int32(0), jnp.int32(0)))

    def padv(t, _):
        sb_src[pl.ds(w + t * 16, 16)] = jnp.zeros((16,), _i32)
        sb_dst[pl.ds(w + t * 16, 16)] = jnp.full((16,), DROW, _i32)
        return 0
    lax.fori_loop(0, 8, padv, 0)
    wr = ((w + CHUNK - 1) // CHUNK) * CHUNK

    def tailflush(f, nch):
        pltpu.sync_copy(sb_src.at[pl.ds(f * CHUNK, CHUNK)],
                        psrc.at[cid, sid, nch])
        pltpu.sync_copy(sb_dst.at[pl.ds(f * CHUNK, CHUNK)],
                        pdst.at[cid, sid, nch])
        return nch + 1
    nch = lax.fori_loop(0, wr // CHUNK, tailflush, nch)

    nst = jnp.maximum((nch + IB - 1) // IB, 12)

    def dpad(p, nch):
        pltpu.sync_copy(d_src, psrc.at[cid, sid, nch])
        pltpu.sync_copy(d_dst, pdst.at[cid, sid, nch])
        return nch + 1
    lax.fori_loop(0, nst * IB - nch, dpad, nch)

    cnt_v[...] = jnp.zeros((16,), _i32) + nst
    pltpu.sync_copy(cnt_v, cnts.at[cid, sid])


@functools.lru_cache(maxsize=None)
def _get_partition():
    return pl.kernel(
        _part_body,
        out_type=(
            jax.ShapeDtypeStruct((2, NS, CAPC, CHUNK), _i32),
            jax.ShapeDtypeStruct((2, NS, CAPC, CHUNK), _i32),
            jax.ShapeDtypeStruct((2, NS, 16), _i32),
        ),
        mesh=plsc.VectorSubcoreMesh(core_axis_name="c", subcore_axis_name="s"),
        scratch_types=[
            pltpu.VMEM((SIB, CHUNK), _i32),
            pltpu.VMEM((SIB, CHUNK), _i32),
            pltpu.VMEM((FB + 160,), _i32),
            pltpu.VMEM((FB + 160,), _i32),
            pltpu.VMEM((CHUNK,), _i32),
            pltpu.VMEM((CHUNK,), _i32),
            pltpu.VMEM((16,), _i32),
        ],
        compiler_params=_SC_PARAMS,
    )


def _sc_body(v, psrc, pdst, cnts, zeros, xn,
             acc, src_blk, dst_blk, rows, cnt_v, sem):
    cid = lax.axis_index("c")
    sid = lax.axis_index("s")

    pltpu.sync_copy(zeros, acc.at[pl.ds(sid * AZ, AZ)])
    pltpu.sync_copy(cnts.at[cid, sid], cnt_v)
    plsc.subcore_barrier()
    ns = 12

    def stage_body(s, carry):
        pltpu.sync_copy(psrc.at[cid, sid, pl.ds(s * IB, IB)], src_blk)
        pltpu.sync_copy(pdst.at[cid, sid, pl.ds(s * IB, IB)], dst_blk)
        pltpu.make_async_copy(v.at[src_blk.at[0]], rows.at[0], sem).start()

        def chunk_body(c, carry):
            cur = lax.rem(c, 2)
            pltpu.make_async_copy(
                v.at[src_blk.at[c]], rows.at[cur], sem).wait()

            @pl.when(c + 1 < IB)
            def _():
                pltpu.make_async_copy(
                    v.at[src_blk.at[c + 1]], rows.at[1 - cur], sem).start()

            pltpu.sync_copy(rows.at[cur], acc.at[dst_blk.at[c]], add=True)
            return carry
        lax.fori_loop(0, IB, chunk_body, 0)
        return carry
    lax.fori_loop(0, ns, stage_body, 0)

    plsc.subcore_barrier()
    pltpu.sync_copy(acc.at[pl.ds(sid * CRPT, CRPT)],
                    xn.at[pl.ds(cid * T + sid * CRPT, CRPT)])


@functools.lru_cache(maxsize=None)
def _get_sc_segsum():
    return pl.kernel(
        _sc_body,
        out_type=jax.ShapeDtypeStruct((NOUT, DM), _f32),
        mesh=plsc.VectorSubcoreMesh(core_axis_name="c", subcore_axis_name="s"),
        scratch_types=[
            pltpu.VMEM_SHARED((AROWS, DM), _f32),
            pltpu.VMEM((IB, CHUNK), _i32),
            pltpu.VMEM((IB, CHUNK), _i32),
            pltpu.VMEM((2, CHUNK, DM), _f32),
            pltpu.VMEM((16,), _i32),
            pltpu.SemaphoreType.DMA,
        ],
        compiler_params=pltpu.CompilerParams(use_tc_tiling_on_sc=False),
    )


def _ln_tc(t, g, b):
    avg = jnp.full((DM, DM), 1.0 / DM, dtype=_f32)
    m = jnp.dot(t, avg, preferred_element_type=_f32)
    c = t - m
    v = jnp.dot(c * c, avg, preferred_element_type=_f32)
    return c * lax.rsqrt(v + 1e-5) * g + b


def _embed_body(nf, wembT, bemb, wvT, xo, vo):
    x = jnp.dot(nf[...], wembT[...], preferred_element_type=_f32) + bemb[...]
    xo[...] = x
    vo[...] = jnp.dot(x, wvT[...], preferred_element_type=_f32)


def _full(shape):
    return pl.BlockSpec(shape, lambda i: (0, 0))


def _rows(width):
    return pl.BlockSpec((ROWBLK, width), lambda i: (i, 0))


_embed_call = pl.pallas_call(
    _embed_body,
    grid=(GRID,),
    in_specs=[_rows(20), _full((20, DM)), _full((1, DM)), _full((DM, DM))],
    out_specs=[_rows(DM), _rows(DM)],
    out_shape=[
        jax.ShapeDtypeStruct((N, DM), _f32),
        jax.ShapeDtypeStruct((N, DM), _f32),
    ],
)


def _layer_body_v(x, xn, g, b, w1T, b1, w2T, b2, wvT, xo, vo):
    t = x[...] + xn[...]
    x1 = _ln_tc(t, g[...], b[...])
    h = jnp.maximum(jnp.dot(x1, w1T[...], preferred_element_type=_f32) + b1[...], 0.0)
    t2 = x1 + jnp.dot(h, w2T[...], preferred_element_type=_f32) + b2[...]
    x2 = _ln_tc(t2, g[...], b[...])
    xo[...] = x2
    vo[...] = jnp.dot(x2, wvT[...], preferred_element_type=_f32)


def _layer_body_last(x, xn, g, b, w1T, b1, w2T, b2, xo):
    t = x[...] + xn[...]
    x1 = _ln_tc(t, g[...], b[...])
    h = jnp.maximum(jnp.dot(x1, w1T[...], preferred_element_type=_f32) + b1[...], 0.0)
    t2 = x1 + jnp.dot(h, w2T[...], preferred_element_type=_f32) + b2[...]
    xo[...] = _ln_tc(t2, g[...], b[...])


_layer_in_specs = [
    _rows(DM), _rows(DM),
    _full((1, DM)), _full((1, DM)),
    _full((DM, 4 * DM)), _full((1, 4 * DM)),
    _full((4 * DM, DM)), _full((1, DM)),
]

_layer_call_v = pl.pallas_call(
    _layer_body_v,
    grid=(GRID,),
    in_specs=_layer_in_specs + [_full((DM, DM))],
    out_specs=[_rows(DM), _rows(DM)],
    out_shape=[
        jax.ShapeDtypeStruct((N, DM), _f32),
        jax.ShapeDtypeStruct((N, DM), _f32),
    ],
)

_layer_call_last = pl.pallas_call(
    _layer_body_last,
    grid=(GRID,),
    in_specs=_layer_in_specs,
    out_specs=[_rows(DM)],
    out_shape=[jax.ShapeDtypeStruct((N, DM), _f32)],
)


@jax.jit
def _run(node_features, edge_index, W_emb, b_emb, Wv, ln_g, ln_b,
         fW1, fb1, fW2, fb2):
    src = edge_index[0]
    dst = edge_index[1]
    pad = EPAD - E
    srcp = jnp.concatenate(
        [src, jnp.zeros((pad,), _i32)]).reshape(NS, CPT, CHUNK)
    dstp = jnp.concatenate(
        [dst, jnp.full((pad,), 2 * NOUT, _i32)]).reshape(NS, CPT, CHUNK)
    zeros = jnp.zeros((AZ, DM), _f32)

    psrc, pdst, cnts = _get_partition()(srcp, dstp)
    x, v = _embed_call(node_features, W_emb.T, b_emb[None, :], Wv[0].T)
    for l in range(L):
        xn = _get_sc_segsum()(v, psrc, pdst, cnts, zeros)
        args = (x, xn, ln_g[l][None, :], ln_b[l][None, :],
                fW1[l].T, fb1[l][None, :], fW2[l].T, fb2[l][None, :])
        if l < L - 1:
            x, v = _layer_call_v(*args, Wv[l + 1].T)
        else:
            (x,) = _layer_call_last(*args)
    return x


def kernel(node_features, edge_index, edge_attr, W_emb, b_emb, Wq, Wk, Wv,
           eW1, eb1, eW2, eb2, ln_g, ln_b, fW1, fb1, fW2, fb2):
    return _run(node_features, edge_index, W_emb, b_emb, Wv, ln_g, ln_b,
                fW1, fb1, fW2, fb2)


# speedup vs baseline: 4.1404x; 4.1404x over previous
import functools

import jax
import jax.numpy as jnp
from jax import lax
from jax.experimental import pallas as pl
from jax.experimental.pallas import tpu as pltpu
from jax.experimental.pallas import tpu_sc as plsc

N = 50000
E = 800000
DM = 64
L = 6

NS = 16
CHUNK = 128

SIB = 28
NSIB = 14
CPT = SIB * NSIB
EPT = CPT * CHUNK
EPAD = NS * EPT

T = 25024
AROWS = 25088
AZ = AROWS // NS
CRPT = T // NS
NOUT = 2 * T

IB = 24
CAPC = 416
FB = 1024
DROW = T

ROWBLK = 2000
GRID = N // ROWBLK

_f32 = jnp.float32
_i32 = jnp.int32

_SC_PARAMS = pltpu.CompilerParams(use_tc_tiling_on_sc=False)


def _part_body(srcp, dstp, psrc, pdst, cnts,
               src_st, dst_st, sb_src, sb_dst, d_src, d_dst, cnt_v):
    cid = lax.axis_index("c")
    sid = lax.axis_index("s")
    base_lo = cid * T

    lane = jax.lax.iota(_i32, 16)
    for t in range(CHUNK // 16):
        d_src[pl.ds(t * 16, 16)] = jnp.zeros((16,), _i32)
        d_dst[pl.ds(t * 16, 16)] = DROW + (t * 16 + lane) % 64

    def scan_stage(s, carry):
        w, nch = carry
        pltpu.sync_copy(srcp.at[sid, pl.ds(s * SIB, SIB)], src_st)
        pltpu.sync_copy(dstp.at[sid, pl.ds(s * SIB, SIB)], dst_st)

        def vreg(i, carry):
            w, nch = carry
            j = i // 8
            o = (i % 8) * 16
            s16 = src_st[j, pl.ds(o, 16)]
            d16 = dst_st[j, pl.ds(o, 16)] - base_lo
            mask = (d16 >= 0) & (d16 < T)
            c = plsc.all_reduce_population_count(mask)[0]
            plsc.store_compressed(sb_src.at[pl.ds(w, 16)], s16, mask=mask)
            plsc.store_compressed(sb_dst.at[pl.ds(w, 16)], d16, mask=mask)
            w = w + c
            do_flush = w >= FB

            @pl.when(do_flush)
            def _():
                def one(f, carry):
                    pltpu.sync_copy(sb_src.at[pl.ds(f * CHUNK, CHUNK)],
                                    psrc.at[cid, sid, nch + f])
                    pltpu.sync_copy(sb_dst.at[pl.ds(f * CHUNK, CHUNK)],
                                    pdst.at[cid, sid, nch + f])
                    return carry
                lax.fori_loop(0, FB // CHUNK, one, 0)
                sb_src[pl.ds(0, 16)] = sb_src[pl.ds(FB, 16)]
                sb_dst[pl.ds(0, 16)] = sb_dst[pl.ds(FB, 16)]

            fl = do_flush.astype(_i32)
            return w - FB * fl, nch + (FB // CHUNK) * fl
        return lax.fori_loop(0, SIB * 8, vreg, (w, nch))

    w, nch = lax.fori_loop(0, NSIB, scan_stage,
                           (jnp.int32(0), jnp.int32(0)))

    def padv(t, _):
        sb_src[pl.ds(w + t * 16, 16)] = jnp.zeros((16,), _i32)
        sb_dst[pl.ds(w + t * 16, 16)] = DROW + (t * 16 + lane) % 64
        return 0
    lax.fori_loop(0, 8, padv, 0)
    wr = ((w + CHUNK - 1) // CHUNK) * CHUNK

    def tailflush(f, nch):
        pltpu.sync_copy(sb_src.at[pl.ds(f * CHUNK, CHUNK)],
                        psrc.at[cid, sid, nch])
        pltpu.sync_copy(sb_dst.at[pl.ds(f * CHUNK, CHUNK)],
                        pdst.at[cid, sid, nch])
        return nch + 1
    nch = lax.fori_loop(0, wr // CHUNK, tailflush, nch)

    nst = (nch + IB - 1) // IB

    def dpad(p, nch):
        pltpu.sync_copy(d_src, psrc.at[cid, sid, nch])
        pltpu.sync_copy(d_dst, pdst.at[cid, sid, nch])
        return nch + 1
    lax.fori_loop(0, nst * IB - nch, dpad, nch)

    cnt_v[...] = jnp.zeros((16,), _i32) + nst
    pltpu.sync_copy(cnt_v, cnts.at[cid, sid])


@functools.lru_cache(maxsize=None)
def _get_partition():
    return pl.kernel(
        _part_body,
        out_type=(
            jax.ShapeDtypeStruct((2, NS, CAPC, CHUNK), _i32),
            jax.ShapeDtypeStruct((2, NS, CAPC, CHUNK), _i32),
            jax.ShapeDtypeStruct((2, NS, 16), _i32),
        ),
        mesh=plsc.VectorSubcoreMesh(core_axis_name="c", subcore_axis_name="s"),
        scratch_types=[
            pltpu.VMEM((SIB, CHUNK), _i32),
            pltpu.VMEM((SIB, CHUNK), _i32),
            pltpu.VMEM((FB + 160,), _i32),
            pltpu.VMEM((FB + 160,), _i32),
            pltpu.VMEM((CHUNK,), _i32),
            pltpu.VMEM((CHUNK,), _i32),
            pltpu.VMEM((16,), _i32),
        ],
        compiler_params=pltpu.CompilerParams(use_tc_tiling_on_sc=False,
                                             needs_layout_passes=False),
    )


def _sc_body(v, psrc, pdst, cnts, zeros, xn,
             acc, src_blk, dst_blk, rows, cnt_v, sem):
    cid = lax.axis_index("c")
    sid = lax.axis_index("s")

    pltpu.sync_copy(zeros, acc.at[pl.ds(sid * AZ, AZ)])
    pltpu.sync_copy(cnts.at[cid, sid], cnt_v)
    plsc.subcore_barrier()
    ns = cnt_v[...][0]

    def stage_body(s, carry):
        pltpu.sync_copy(psrc.at[cid, sid, pl.ds(s * IB, IB)], src_blk)
        pltpu.sync_copy(pdst.at[cid, sid, pl.ds(s * IB, IB)], dst_blk)
        pltpu.make_async_copy(v.at[src_blk.at[0]], rows.at[0], sem).start()

        def chunk_body(c, carry):
            cur = lax.rem(c, 2)
            pltpu.make_async_copy(
                v.at[src_blk.at[c]], rows.at[cur], sem).wait()

            @pl.when(c + 1 < IB)
            def _():
                pltpu.make_async_copy(
                    v.at[src_blk.at[c + 1]], rows.at[1 - cur], sem).start()

            pltpu.sync_copy(rows.at[cur], acc.at[dst_blk.at[c]], add=True)
            return carry
        lax.fori_loop(0, IB, chunk_body, 0)
        return carry
    lax.fori_loop(0, ns, stage_body, 0)

    plsc.subcore_barrier()
    pltpu.sync_copy(acc.at[pl.ds(sid * CRPT, CRPT)],
                    xn.at[pl.ds(cid * T + sid * CRPT, CRPT)])


@functools.lru_cache(maxsize=None)
def _get_sc_segsum():
    return pl.kernel(
        _sc_body,
        out_type=jax.ShapeDtypeStruct((NOUT, DM), _f32),
        mesh=plsc.VectorSubcoreMesh(core_axis_name="c", subcore_axis_name="s"),
        scratch_types=[
            pltpu.VMEM_SHARED((AROWS, DM), _f32),
            pltpu.VMEM((IB, CHUNK), _i32),
            pltpu.VMEM((IB, CHUNK), _i32),
            pltpu.VMEM((2, CHUNK, DM), _f32),
            pltpu.VMEM((16,), _i32),
            pltpu.SemaphoreType.DMA,
        ],
        compiler_params=pltpu.CompilerParams(use_tc_tiling_on_sc=False),
    )


def _ln_tc(t, g, b):
    avg = jnp.full((DM, DM), 1.0 / DM, dtype=_f32)
    m = jnp.dot(t, avg, preferred_element_type=_f32)
    c = t - m
    v = jnp.dot(c * c, avg, preferred_element_type=_f32)
    return c * lax.rsqrt(v + 1e-5) * g + b


def _embed_body(nf, wembT, bemb, wvT, xo, vo):
    x = jnp.dot(nf[...], wembT[...], preferred_element_type=_f32) + bemb[...]
    xo[...] = x
    vo[...] = jnp.dot(x, wvT[...], preferred_element_type=_f32)


def _full(shape):
    return pl.BlockSpec(shape, lambda i: (0, 0))


def _rows(width):
    return pl.BlockSpec((ROWBLK, width), lambda i: (i, 0))


_embed_call = pl.pallas_call(
    _embed_body,
    grid=(GRID,),
    in_specs=[_rows(20), _full((20, DM)), _full((1, DM)), _full((DM, DM))],
    out_specs=[_rows(DM), _rows(DM)],
    out_shape=[
        jax.ShapeDtypeStruct((N, DM), _f32),
        jax.ShapeDtypeStruct((N, DM), _f32),
    ],
)


def _layer_body_v(x, xn, g, b, w1T, b1, w2T, b2, wvT, xo, vo):
    t = x[...] + xn[...]
    x1 = _ln_tc(t, g[...], b[...])
    h = jnp.maximum(jnp.dot(x1, w1T[...], preferred_element_type=_f32) + b1[...], 0.0)
    t2 = x1 + jnp.dot(h, w2T[...], preferred_element_type=_f32) + b2[...]
    x2 = _ln_tc(t2, g[...], b[...])
    xo[...] = x2
    vo[...] = jnp.dot(x2, wvT[...], preferred_element_type=_f32)


def _layer_body_last(x, xn, g, b, w1T, b1, w2T, b2, xo):
    t = x[...] + xn[...]
    x1 = _ln_tc(t, g[...], b[...])
    h = jnp.maximum(jnp.dot(x1, w1T[...], preferred_element_type=_f32) + b1[...], 0.0)
    t2 = x1 + jnp.dot(h, w2T[...], preferred_element_type=_f32) + b2[...]
    xo[...] = _ln_tc(t2, g[...], b[...])


_layer_in_specs = [
    _rows(DM), _rows(DM),
    _full((1, DM)), _full((1, DM)),
    _full((DM, 4 * DM)), _full((1, 4 * DM)),
    _full((4 * DM, DM)), _full((1, DM)),
]

_layer_call_v = pl.pallas_call(
    _layer_body_v,
    grid=(GRID,),
    in_specs=_layer_in_specs + [_full((DM, DM))],
    out_specs=[_rows(DM), _rows(DM)],
    out_shape=[
        jax.ShapeDtypeStruct((N, DM), _f32),
        jax.ShapeDtypeStruct((N, DM), _f32),
    ],
)

_layer_call_last = pl.pallas_call(
    _layer_body_last,
    grid=(GRID,),
    in_specs=_layer_in_specs,
    out_specs=[_rows(DM)],
    out_shape=[jax.ShapeDtypeStruct((N, DM), _f32)],
)


@jax.jit
def _run(node_features, edge_index, W_emb, b_emb, Wv, ln_g, ln_b,
         fW1, fb1, fW2, fb2):
    src = edge_index[0]
    dst = edge_index[1]
    pad = EPAD - E
    srcp = jnp.concatenate(
        [src, jnp.zeros((pad,), _i32)]).reshape(NS, CPT, CHUNK)
    dstp = jnp.concatenate(
        [dst, jnp.full((pad,), 2 * NOUT, _i32)]).reshape(NS, CPT, CHUNK)
    zeros = jnp.zeros((AZ, DM), _f32)

    psrc, pdst, cnts = _get_partition()(srcp, dstp)
    x, v = _embed_call(node_features, W_emb.T, b_emb[None, :], Wv[0].T)
    for l in range(L):
        xn = _get_sc_segsum()(v, psrc, pdst, cnts, zeros)
        args = (x, xn, ln_g[l][None, :], ln_b[l][None, :],
                fW1[l].T, fb1[l][None, :], fW2[l].T, fb2[l][None, :])
        if l < L - 1:
            x, v = _layer_call_v(*args, Wv[l + 1].T)
        else:
            (x,) = _layer_call_last(*args)
    return x


def kernel(node_features, edge_index, edge_attr, W_emb, b_emb, Wq, Wk, Wv,
           eW1, eb1, eW2, eb2, ln_g, ln_b, fW1, fb1, fW2, fb2):
    return _run(node_features, edge_index, W_emb, b_emb, Wv, ln_g, ln_b,
                fW1, fb1, fW2, fb2)
